# baseline (speedup 1.0000x reference)
import functools

import jax
import jax.numpy as jnp
from jax import lax
from jax.experimental import pallas as pl
from jax.experimental.pallas import tpu as pltpu
from jax.experimental.pallas import tpu_sc as plsc

B = 16384
EMBED_DIM = 128
HIDDEN = 2048

BCS = (2048, 2048, 4096, 8192)

NUM_CORES = 2
NUM_SUBCORES = 16
NW = NUM_CORES * NUM_SUBCORES


def _sc_gather(table, xt, bc):
    bpw = bc // NW
    mesh = plsc.VectorSubcoreMesh(core_axis_name="c", subcore_axis_name="s")

    @functools.partial(
        pl.kernel,
        mesh=mesh,
        out_type=jax.ShapeDtypeStruct((4, bc, EMBED_DIM), jnp.float32),
        scratch_types=[
            pltpu.VMEM((4 * bpw,), jnp.int32),
            pltpu.VMEM((bpw, EMBED_DIM), jnp.float32),
            pltpu.VMEM((bpw, EMBED_DIM), jnp.float32),
            pltpu.SemaphoreType.DMA,
            pltpu.SemaphoreType.DMA,
            pltpu.SemaphoreType.DMA,
            pltpu.SemaphoreType.DMA,
        ],
    )
    def k(table_hbm, xt_hbm, out_hbm, idx_v, rows0, rows1, g0, g1, s0, s1):
        wid = lax.axis_index("s") * NUM_CORES + lax.axis_index("c")
        base = wid * bpw
        for c in range(4):
            pltpu.sync_copy(xt_hbm.at[c, pl.ds(base, bpw)],
                            idx_v.at[pl.ds(c * bpw, bpw)])
        bufs = (rows0, rows1)
        gsems = (g0, g1)
        ssems = (s0, s1)
        gathers = [None, None]
        scatters = [None, None]
        for it in range(4):
            s = it % 2
            if it >= 2:
                scatters[s].wait()
            gathers[s] = pltpu.async_copy(
                table_hbm.at[idx_v.at[pl.ds(it * bpw, bpw)]],
                bufs[s], gsems[s])
            if it >= 1:
                p = (it - 1) % 2
                gathers[p].wait()
                scatters[p] = pltpu.async_copy(
                    bufs[p], out_hbm.at[it - 1, pl.ds(base, bpw)], ssems[p])
        gathers[1].wait()
        scatters[1] = pltpu.async_copy(
            bufs[1], out_hbm.at[3, pl.ds(base, bpw)], ssems[1])
        scatters[0].wait()
        scatters[1].wait()

    return k(table, xt)


def _cast_pair_bf16(Wa, Wb):
    R = 256

    def body(a_ref, b_ref, oa_ref, ob_ref):
        oa_ref[...] = a_ref[...].astype(jnp.bfloat16)
        ob_ref[...] = b_ref[...].astype(jnp.bfloat16)

    spec = pl.BlockSpec((R, HIDDEN), lambda i: (i, 0))
    out = jax.ShapeDtypeStruct((HIDDEN, HIDDEN), jnp.bfloat16)
    return pl.pallas_call(
        body,
        grid=(HIDDEN // R,),
        in_specs=[spec, spec],
        out_specs=[spec, spec],
        out_shape=[out, out],
    )(Wa, Wb)


TB = 1024


def _mlp_body(g_ref, w0_ref, b0_ref, w1_ref, b1_ref, w2_ref, b2_ref,
              wout_ref, bout_ref, y_ref):
    bf = jnp.bfloat16
    g = jnp.concatenate(
        [g_ref[0], g_ref[1], g_ref[2], g_ref[3]], axis=1).astype(bf)
    h = jnp.dot(g, w0_ref[...], preferred_element_type=jnp.float32)
    h = jnp.maximum(h + b0_ref[...], 0.0).astype(bf)
    h = jnp.dot(h, w1_ref[...], preferred_element_type=jnp.float32)
    h = jnp.maximum(h + b1_ref[...], 0.0).astype(bf)
    h = jnp.dot(h, w2_ref[...], preferred_element_type=jnp.float32)
    h = jnp.maximum(h + b2_ref[...], 0.0).astype(bf)
    y = jnp.dot(h, wout_ref[...], preferred_element_type=jnp.float32)
    y_ref[...] = y + bout_ref[...]


def _mlp(g4, W0b, b0, W1b, b1, W2b, b2, Woutb, bout, bc):
    nb = bc // TB
    full = lambda shape: pl.BlockSpec(shape, lambda i: (0, 0))
    return pl.pallas_call(
        _mlp_body,
        grid=(nb,),
        in_specs=[
            pl.BlockSpec((4, TB, EMBED_DIM), lambda i: (0, i, 0)),
            full((4 * EMBED_DIM, HIDDEN)),
            full((1, HIDDEN)),
            full((HIDDEN, HIDDEN)),
            full((1, HIDDEN)),
            full((HIDDEN, HIDDEN)),
            full((1, HIDDEN)),
            full((HIDDEN, 1)),
            full((1, 1)),
        ],
        out_specs=pl.BlockSpec((TB, 1), lambda i: (i, 0)),
        out_shape=jax.ShapeDtypeStruct((bc, 1), jnp.float32),
        compiler_params=pltpu.CompilerParams(
            dimension_semantics=("arbitrary",),
        ),
    )(g4, W0b, b0, W1b, b1, W2b, b2, Woutb, bout)


def kernel(x, aug_table, mag_table, W0, b0, W1, b1, W2, b2, Wout, bout):
    del mag_table
    bf = jnp.bfloat16
    xt = x.T.astype(jnp.int32)
    W1b, W2b = _cast_pair_bf16(W1, W2)
    W0b = W0.astype(bf)
    Woutb = Wout.astype(bf)
    b0r = b0.reshape(1, HIDDEN)
    b1r = b1.reshape(1, HIDDEN)
    b2r = b2.reshape(1, HIDDEN)
    boutr = bout.reshape(1, 1)
    gs = []
    off = 0
    for bc in BCS:
        gs.append(_sc_gather(aug_table, xt[:, off:off + bc], bc))
        off += bc
    ys = [_mlp(g, W0b, b0r, W1b, b1r, W2b, b2r, Woutb, boutr, bc)
          for g, bc in zip(gs, BCS)]
    return jnp.concatenate(ys, axis=0)


# speedup vs baseline: 1.0618x; 1.0618x over previous
import functools

import jax
import jax.numpy as jnp
from jax import lax
from jax.experimental import pallas as pl
from jax.experimental.pallas import tpu as pltpu
from jax.experimental.pallas import tpu_sc as plsc

B = 16384
EMBED_DIM = 128
HIDDEN = 2048

BCS = (8192, 8192)

NUM_CORES = 2
NUM_SUBCORES = 16
NW = NUM_CORES * NUM_SUBCORES


def _sc_gather(table, xt, bc):
    bpw = bc // NW
    mesh = plsc.VectorSubcoreMesh(core_axis_name="c", subcore_axis_name="s")

    @functools.partial(
        pl.kernel,
        mesh=mesh,
        out_type=jax.ShapeDtypeStruct((4, bc, EMBED_DIM), jnp.float32),
        scratch_types=[
            pltpu.VMEM((4 * bpw,), jnp.int32),
            pltpu.VMEM((bpw, EMBED_DIM), jnp.float32),
            pltpu.VMEM((bpw, EMBED_DIM), jnp.float32),
            pltpu.SemaphoreType.DMA,
            pltpu.SemaphoreType.DMA,
            pltpu.SemaphoreType.DMA,
            pltpu.SemaphoreType.DMA,
        ],
    )
    def k(table_hbm, xt_hbm, out_hbm, idx_v, rows0, rows1, g0, g1, s0, s1):
        wid = lax.axis_index("s") * NUM_CORES + lax.axis_index("c")
        base = wid * bpw
        for c in range(4):
            pltpu.sync_copy(xt_hbm.at[c, pl.ds(base, bpw)],
                            idx_v.at[pl.ds(c * bpw, bpw)])
        bufs = (rows0, rows1)
        gsems = (g0, g1)
        ssems = (s0, s1)
        gathers = [None, None]
        scatters = [None, None]
        for it in range(4):
            s = it % 2
            if it >= 2:
                scatters[s].wait()
            gathers[s] = pltpu.async_copy(
                table_hbm.at[idx_v.at[pl.ds(it * bpw, bpw)]],
                bufs[s], gsems[s])
            if it >= 1:
                p = (it - 1) % 2
                gathers[p].wait()
                scatters[p] = pltpu.async_copy(
                    bufs[p], out_hbm.at[it - 1, pl.ds(base, bpw)], ssems[p])
        gathers[1].wait()
        scatters[1] = pltpu.async_copy(
            bufs[1], out_hbm.at[3, pl.ds(base, bpw)], ssems[1])
        scatters[0].wait()
        scatters[1].wait()

    return k(table, xt)


TB = 512


def _mlp_body(g_ref, w0_ref, b0_ref, w1_ref, b1_ref, w2_ref, b2_ref,
              wout_ref, bout_ref, y_ref):
    g = jnp.concatenate([g_ref[0], g_ref[1], g_ref[2], g_ref[3]], axis=1)
    h = jnp.dot(g, w0_ref[...], preferred_element_type=jnp.float32)
    h = jnp.maximum(h + b0_ref[...], 0.0)
    h = jnp.dot(h, w1_ref[...], preferred_element_type=jnp.float32)
    h = jnp.maximum(h + b1_ref[...], 0.0)
    h = jnp.dot(h, w2_ref[...], preferred_element_type=jnp.float32)
    h = jnp.maximum(h + b2_ref[...], 0.0)
    y = jnp.dot(h, wout_ref[...], preferred_element_type=jnp.float32)
    y_ref[...] = y + bout_ref[...]


def _mlp(g4, W0, b0, W1, b1, W2, b2, Wout, bout, bc):
    nb = bc // TB
    full = lambda shape: pl.BlockSpec(shape, lambda i: (0, 0))
    return pl.pallas_call(
        _mlp_body,
        grid=(nb,),
        in_specs=[
            pl.BlockSpec((4, TB, EMBED_DIM), lambda i: (0, i, 0)),
            full((4 * EMBED_DIM, HIDDEN)),
            full((1, HIDDEN)),
            full((HIDDEN, HIDDEN)),
            full((1, HIDDEN)),
            full((HIDDEN, HIDDEN)),
            full((1, HIDDEN)),
            full((HIDDEN, 1)),
            full((1, 1)),
        ],
        out_specs=pl.BlockSpec((TB, 1), lambda i: (i, 0)),
        out_shape=jax.ShapeDtypeStruct((bc, 1), jnp.float32),
        compiler_params=pltpu.CompilerParams(
            dimension_semantics=("arbitrary",),
        ),
    )(g4, W0, b0, W1, b1, W2, b2, Wout, bout)


def kernel(x, aug_table, mag_table, W0, b0, W1, b1, W2, b2, Wout, bout):
    del mag_table
    xt = x.T.astype(jnp.int32)
    b0r = b0.reshape(1, HIDDEN)
    b1r = b1.reshape(1, HIDDEN)
    b2r = b2.reshape(1, HIDDEN)
    boutr = bout.reshape(1, 1)
    gs = []
    off = 0
    for bc in BCS:
        gs.append(_sc_gather(aug_table, xt[:, off:off + bc], bc))
        off += bc
    ys = [_mlp(g, W0, b0r, W1, b1r, W2, b2r, Wout, boutr, bc)
          for g, bc in zip(gs, BCS)]
    return jnp.concatenate(ys, axis=0)
